# baseline (speedup 1.0000x reference)
import jax
import jax.numpy as jnp
import numpy as np
from jax import lax
from jax.experimental import pallas as pl
from jax.experimental.pallas import tpu as pltpu
from jax.experimental.pallas import tpu_sc as plsc

N = 10000
E = 160000
D_IN = 32
D_OUT = 32
D_EDGE = 16
HID = 1024
W48 = 48

NC = 2
NS = 16
NW = NC * NS

EH = E // 2
ROWS_W = EH // NW
CHUNK = 125
CHUNKS_W = ROWS_W // CHUNK
OUTER = 5
INNER = CHUNKS_W // OUTER


def _mesh():
    return plsc.VectorSubcoreMesh(core_axis_name="c", subcore_axis_name="s",
                                  num_cores=NC, num_subcores=NS)


def _gather_body(x_hbm, srcm_hbm, out_hbm, idx_v, row_v, sem):
    wid = lax.axis_index("s") * NC + lax.axis_index("c")
    pltpu.sync_copy(srcm_hbm.at[pl.ds(wid * CHUNKS_W, CHUNKS_W)], idx_v)

    def body(j, carry):
        pltpu.async_copy(x_hbm.at[idx_v.at[j]], row_v, sem).wait()
        pltpu.sync_copy(row_v,
                        out_hbm.at[pl.ds(wid * ROWS_W + j * CHUNK, CHUNK)])
        return carry

    lax.fori_loop(0, CHUNKS_W, body, 0)


@jax.jit
def _sc_gather(x, src_m):
    return pl.kernel(
        _gather_body,
        out_type=jax.ShapeDtypeStruct((EH, D_IN), jnp.float32),
        mesh=_mesh(),
        scratch_types=[
            pltpu.VMEM((CHUNKS_W, CHUNK), jnp.int32),
            pltpu.VMEM((CHUNK, D_IN), jnp.float32),
            pltpu.SemaphoreType.DMA,
        ],
        compiler_params=pltpu.CompilerParams(use_tc_tiling_on_sc=False),
    )(x, src_m)


def _scatter_body(msg_hbm, dstm_hbm, zero_hbm, out0_hbm, out1_hbm,
                  idx_v, msg_v, acc_sh):
    cid = lax.axis_index("c")
    sid = lax.axis_index("s")
    wid = sid * NC + cid

    @pl.when(sid == 0)
    def _():
        pltpu.sync_copy(zero_hbm, acc_sh)

    plsc.subcore_barrier()

    pltpu.sync_copy(dstm_hbm.at[pl.ds(wid * CHUNKS_W, CHUNKS_W)], idx_v)

    def outer(c, carry):
        pltpu.sync_copy(
            msg_hbm.at[pl.ds(wid * ROWS_W + c * (INNER * CHUNK),
                             INNER * CHUNK)], msg_v)

        def inner(j, carry2):
            pltpu.sync_copy(msg_v.at[pl.ds(j * CHUNK, CHUNK)],
                            acc_sh.at[idx_v.at[c * INNER + j]], add=True)
            return carry2

        lax.fori_loop(0, INNER, inner, 0)
        return carry

    lax.fori_loop(0, OUTER, outer, 0)
    plsc.subcore_barrier()

    rows0 = sid * (N // NS)

    @pl.when(cid == 0)
    def _():
        pltpu.sync_copy(acc_sh.at[pl.ds(rows0, N // NS)],
                        out0_hbm.at[pl.ds(rows0, N // NS)])

    @pl.when(cid == 1)
    def _():
        pltpu.sync_copy(acc_sh.at[pl.ds(rows0, N // NS)],
                        out1_hbm.at[pl.ds(rows0, N // NS)])


@jax.jit
def _sc_scatter(msg, dst_m, zeros_acc):
    return pl.kernel(
        _scatter_body,
        out_type=(jax.ShapeDtypeStruct((N, W48), jnp.float32),
                  jax.ShapeDtypeStruct((N, W48), jnp.float32)),
        mesh=_mesh(),
        scratch_types=[
            pltpu.VMEM((CHUNKS_W, CHUNK), jnp.int32),
            pltpu.VMEM((INNER * CHUNK, W48), jnp.float32),
            pltpu.VMEM_SHARED((N, W48), jnp.float32),
        ],
        compiler_params=pltpu.CompilerParams(use_tc_tiling_on_sc=False),
    )(msg, dst_m, zeros_acc)


TE = 2000


def _edge_body(ea_ref, xj_ref, w1_ref, b1_ref, w2_ref,
               r_ref, s_ref, b2m_ref, c_ref, out_ref):
    h = jnp.dot(ea_ref[...].astype(jnp.bfloat16), w1_ref[...],
                preferred_element_type=jnp.float32)
    h = h + b1_ref[...]
    h = jnp.maximum(h, 0.01 * h)
    w = jnp.dot(h.astype(jnp.bfloat16), w2_ref[...],
                preferred_element_type=jnp.float32)
    xjb = xj_ref[...].astype(jnp.bfloat16)
    xr = jnp.dot(xjb, r_ref[...],
                 preferred_element_type=jnp.float32)
    msg = jnp.dot((xr * w).astype(jnp.bfloat16), s_ref[...],
                  preferred_element_type=jnp.float32)
    out_ref[...] = msg + jnp.dot(xjb, b2m_ref[...],
                                 preferred_element_type=jnp.float32) + c_ref[...]


_R_EXPAND = np.zeros((D_IN, HID), dtype=np.float32)
for _i in range(D_IN):
    _R_EXPAND[_i, _i * D_OUT:(_i + 1) * D_OUT] = 1.0
_S_SELECT = np.zeros((HID, W48), dtype=np.float32)
for _i in range(D_IN):
    for _o in range(D_OUT):
        _S_SELECT[_i * D_OUT + _o, _o] = 1.0
_C_ONES = np.zeros((1, W48), dtype=np.float32)
_C_ONES[0, D_OUT] = 1.0


@jax.jit
def _tc_edge(ea, xj, W1, b1, W2, b2):
    grid = (EH // TE,)
    return pl.pallas_call(
        _edge_body,
        grid=grid,
        in_specs=[
            pl.BlockSpec((TE, D_EDGE), lambda i: (i, 0)),
            pl.BlockSpec((TE, D_IN), lambda i: (i, 0)),
            pl.BlockSpec((D_EDGE, HID), lambda i: (0, 0)),
            pl.BlockSpec((1, HID), lambda i: (0, 0)),
            pl.BlockSpec((HID, HID), lambda i: (0, 0)),
            pl.BlockSpec((D_IN, HID), lambda i: (0, 0)),
            pl.BlockSpec((HID, W48), lambda i: (0, 0)),
            pl.BlockSpec((D_IN, W48), lambda i: (0, 0)),
            pl.BlockSpec((1, W48), lambda i: (0, 0)),
        ],
        out_specs=pl.BlockSpec((TE, W48), lambda i: (i, 0)),
        out_shape=jax.ShapeDtypeStruct((EH, W48), jnp.float32),
        compiler_params=pltpu.CompilerParams(
            dimension_semantics=("arbitrary",)),
    )(ea, xj, W1.astype(jnp.bfloat16),
      b1.reshape(1, HID),
      W2.astype(jnp.bfloat16),
      jnp.asarray(_R_EXPAND, dtype=jnp.bfloat16),
      jnp.asarray(_S_SELECT, dtype=jnp.bfloat16),
      jnp.pad(b2.reshape(D_IN, D_OUT),
              ((0, 0), (0, W48 - D_OUT))).astype(jnp.bfloat16),
      jnp.asarray(_C_ONES))


def _finish_body(p0_ref, p1_ref, p2_ref, p3_ref, x_ref, root_ref, cb_ref,
                 g_ref, b_ref, out_ref):
    p = (p0_ref[...] + p1_ref[...]) + (p2_ref[...] + p3_ref[...])
    agg = p[:, :D_OUT] / jnp.maximum(p[:, D_OUT:D_OUT + 1], 1.0)
    pre = agg + jnp.dot(x_ref[...], root_ref[...],
                        preferred_element_type=jnp.float32) + cb_ref[...]
    mean = jnp.mean(pre, axis=0, keepdims=True)
    cen = pre - mean
    var = jnp.mean(cen * cen, axis=0, keepdims=True)
    out_ref[...] = cen * lax.rsqrt(var + 1e-5) * g_ref[...] + b_ref[...]


@jax.jit
def _tc_finish(p0, p1, p2, p3, x, root, conv_bias, gamma, beta):
    return pl.pallas_call(
        _finish_body,
        out_shape=jax.ShapeDtypeStruct((N, D_OUT), jnp.float32),
    )(p0, p1, p2, p3, x, root, conv_bias.reshape(1, D_OUT),
      gamma.reshape(1, D_OUT), beta.reshape(1, D_OUT))


def kernel(x, edge_index, edge_attr, W1, b1, W2, b2, root, conv_bias,
           gamma, beta):
    src = edge_index[0].astype(jnp.int32)
    dst = edge_index[1].astype(jnp.int32)
    s0 = src[:EH].reshape(NW * CHUNKS_W, CHUNK)
    s1 = src[EH:].reshape(NW * CHUNKS_W, CHUNK)
    d0 = dst[:EH].reshape(NW * CHUNKS_W, CHUNK)
    d1 = dst[EH:].reshape(NW * CHUNKS_W, CHUNK)
    zeros_acc = jnp.zeros((N, W48), jnp.float32)

    xj0 = _sc_gather(x, s0)
    xj1 = _sc_gather(x, s1)
    msg0 = _tc_edge(edge_attr[:EH], xj0, W1, b1, W2, b2)
    p00, p01 = _sc_scatter(msg0, d0, zeros_acc)
    msg1 = _tc_edge(edge_attr[EH:], xj1, W1, b1, W2, b2)
    p10, p11 = _sc_scatter(msg1, d1, zeros_acc)
    return _tc_finish(p00, p01, p10, p11, x, root, conv_bias, gamma, beta)


# speedup vs baseline: 1.0630x; 1.0630x over previous
import jax
import jax.numpy as jnp
import numpy as np
from jax import lax
from jax.experimental import pallas as pl
from jax.experimental.pallas import tpu as pltpu
from jax.experimental.pallas import tpu_sc as plsc

N = 10000
E = 160000
D_IN = 32
D_OUT = 32
D_EDGE = 16
HID = 1024
WPAD = 128
CCOL = D_OUT

NC = 2
NS = 16
NW = NC * NS

ROWS_W = E // NW
CHUNK = 125
CHUNKS_W = ROWS_W // CHUNK
OUTER = 20
INNER = CHUNKS_W // OUTER


def _mesh():
    return plsc.VectorSubcoreMesh(core_axis_name="c", subcore_axis_name="s",
                                  num_cores=NC, num_subcores=NS)


def _gather_body(x_hbm, srcm_hbm, out_hbm, idx_v, row_v, sem):
    wid = lax.axis_index("s") * NC + lax.axis_index("c")
    pltpu.sync_copy(srcm_hbm.at[pl.ds(wid * CHUNKS_W, CHUNKS_W)], idx_v)

    def body(j, carry):
        pltpu.async_copy(x_hbm.at[idx_v.at[j]], row_v, sem).wait()
        pltpu.sync_copy(row_v,
                        out_hbm.at[pl.ds(wid * ROWS_W + j * CHUNK, CHUNK)])
        return carry

    lax.fori_loop(0, CHUNKS_W, body, 0)


@jax.jit
def _sc_gather(x128, src_m):
    return pl.kernel(
        _gather_body,
        out_type=jax.ShapeDtypeStruct((E, WPAD), jnp.float32),
        mesh=_mesh(),
        scratch_types=[
            pltpu.VMEM((CHUNKS_W, CHUNK), jnp.int32),
            pltpu.VMEM((CHUNK, WPAD), jnp.float32),
            pltpu.SemaphoreType.DMA,
        ],
        compiler_params=pltpu.CompilerParams(use_tc_tiling_on_sc=False),
    )(x128, src_m)


def _scatter_body(msg_hbm, dstm_hbm, zero_hbm, out0_hbm, out1_hbm,
                  idx_v, msg_v, acc_sh):
    cid = lax.axis_index("c")
    sid = lax.axis_index("s")
    wid = sid * NC + cid

    @pl.when(sid == 0)
    def _():
        pltpu.sync_copy(zero_hbm, acc_sh)

    plsc.subcore_barrier()

    pltpu.sync_copy(dstm_hbm.at[pl.ds(wid * CHUNKS_W, CHUNKS_W)], idx_v)

    def outer(c, carry):
        pltpu.sync_copy(
            msg_hbm.at[pl.ds(wid * ROWS_W + c * (INNER * CHUNK),
                             INNER * CHUNK)], msg_v)

        def inner(j, carry2):
            pltpu.sync_copy(msg_v.at[pl.ds(j * CHUNK, CHUNK)],
                            acc_sh.at[idx_v.at[c * INNER + j]], add=True)
            return carry2

        lax.fori_loop(0, INNER, inner, 0)
        return carry

    lax.fori_loop(0, OUTER, outer, 0)
    plsc.subcore_barrier()

    rows0 = sid * (N // NS)

    @pl.when(cid == 0)
    def _():
        pltpu.sync_copy(acc_sh.at[pl.ds(rows0, N // NS)],
                        out0_hbm.at[pl.ds(rows0, N // NS)])

    @pl.when(cid == 1)
    def _():
        pltpu.sync_copy(acc_sh.at[pl.ds(rows0, N // NS)],
                        out1_hbm.at[pl.ds(rows0, N // NS)])


@jax.jit
def _sc_scatter(msg, dst_m, zeros_acc):
    return pl.kernel(
        _scatter_body,
        out_type=(jax.ShapeDtypeStruct((N, WPAD), jnp.float32),
                  jax.ShapeDtypeStruct((N, WPAD), jnp.float32)),
        mesh=_mesh(),
        scratch_types=[
            pltpu.VMEM((CHUNKS_W, CHUNK), jnp.int32),
            pltpu.VMEM((INNER * CHUNK, WPAD), jnp.float32),
            pltpu.VMEM_SHARED((N, WPAD), jnp.float32),
        ],
        compiler_params=pltpu.CompilerParams(use_tc_tiling_on_sc=False),
    )(msg, dst_m, zeros_acc)


TE = 2000


def _edge_body(ea_ref, xj_ref, w1_ref, b1_ref, w2_ref,
               r_ref, s_ref, b2m_ref, c_ref, out_ref):
    h = jnp.dot(ea_ref[...].astype(jnp.bfloat16), w1_ref[...],
                preferred_element_type=jnp.float32)
    h = h + b1_ref[...]
    h = jnp.maximum(h, 0.01 * h)
    w = jnp.dot(h.astype(jnp.bfloat16), w2_ref[...],
                preferred_element_type=jnp.float32)
    xjb = xj_ref[:, :D_IN].astype(jnp.bfloat16)
    xr = jnp.dot(xjb, r_ref[...],
                 preferred_element_type=jnp.float32)
    msg = jnp.dot((xr * w).astype(jnp.bfloat16), s_ref[...],
                  preferred_element_type=jnp.float32)
    out_ref[...] = msg + jnp.dot(xjb, b2m_ref[...],
                                 preferred_element_type=jnp.float32) + c_ref[...]


_R_EXPAND = np.zeros((D_IN, HID), dtype=np.float32)
for _i in range(D_IN):
    _R_EXPAND[_i, _i * D_OUT:(_i + 1) * D_OUT] = 1.0
_S_SELECT = np.zeros((HID, WPAD), dtype=np.float32)
for _i in range(D_IN):
    for _o in range(D_OUT):
        _S_SELECT[_i * D_OUT + _o, _o] = 1.0
_C_ONES = np.zeros((1, WPAD), dtype=np.float32)
_C_ONES[0, CCOL] = 1.0


@jax.jit
def _tc_edge(ea, xj, W1, b1, W2, b2):
    grid = (E // TE,)
    return pl.pallas_call(
        _edge_body,
        grid=grid,
        in_specs=[
            pl.BlockSpec((TE, D_EDGE), lambda i: (i, 0)),
            pl.BlockSpec((TE, WPAD), lambda i: (i, 0)),
            pl.BlockSpec((D_EDGE, HID), lambda i: (0, 0)),
            pl.BlockSpec((1, HID), lambda i: (0, 0)),
            pl.BlockSpec((HID, HID), lambda i: (0, 0)),
            pl.BlockSpec((D_IN, HID), lambda i: (0, 0)),
            pl.BlockSpec((HID, WPAD), lambda i: (0, 0)),
            pl.BlockSpec((D_IN, WPAD), lambda i: (0, 0)),
            pl.BlockSpec((1, WPAD), lambda i: (0, 0)),
        ],
        out_specs=pl.BlockSpec((TE, WPAD), lambda i: (i, 0)),
        out_shape=jax.ShapeDtypeStruct((E, WPAD), jnp.float32),
        compiler_params=pltpu.CompilerParams(
            dimension_semantics=("arbitrary",)),
    )(ea, xj, W1.astype(jnp.bfloat16),
      b1.reshape(1, HID),
      W2.astype(jnp.bfloat16),
      jnp.asarray(_R_EXPAND, dtype=jnp.bfloat16),
      jnp.asarray(_S_SELECT, dtype=jnp.bfloat16),
      jnp.pad(b2.reshape(D_IN, D_OUT),
              ((0, 0), (0, WPAD - D_OUT))).astype(jnp.bfloat16),
      jnp.asarray(_C_ONES))


def _finish_body(p0_ref, p1_ref, x_ref, root_ref, cb_ref, g_ref, b_ref,
                 out_ref):
    p = p0_ref[...] + p1_ref[...]
    agg = p[:, :D_OUT] / jnp.maximum(p[:, CCOL:CCOL + 1], 1.0)
    pre = agg + jnp.dot(x_ref[...], root_ref[...],
                        preferred_element_type=jnp.float32) + cb_ref[...]
    mean = jnp.mean(pre, axis=0, keepdims=True)
    cen = pre - mean
    var = jnp.mean(cen * cen, axis=0, keepdims=True)
    out_ref[...] = cen * lax.rsqrt(var + 1e-5) * g_ref[...] + b_ref[...]


@jax.jit
def _tc_finish(p0, p1, x, root, conv_bias, gamma, beta):
    return pl.pallas_call(
        _finish_body,
        out_shape=jax.ShapeDtypeStruct((N, D_OUT), jnp.float32),
    )(p0, p1, x, root, conv_bias.reshape(1, D_OUT),
      gamma.reshape(1, D_OUT), beta.reshape(1, D_OUT))


def kernel(x, edge_index, edge_attr, W1, b1, W2, b2, root, conv_bias,
           gamma, beta):
    src_m = edge_index[0].astype(jnp.int32).reshape(NW * CHUNKS_W, CHUNK)
    dst_m = edge_index[1].astype(jnp.int32).reshape(NW * CHUNKS_W, CHUNK)
    x128 = jnp.pad(x, ((0, 0), (0, WPAD - D_IN)))

    xj = _sc_gather(x128, src_m)
    msg = _tc_edge(edge_attr, xj, W1, b1, W2, b2)
    zeros_acc = jnp.zeros((N, WPAD), jnp.float32)
    p0, p1 = _sc_scatter(msg, dst_m, zeros_acc)
    return _tc_finish(p0, p1, x, root, conv_bias, gamma, beta)


# speedup vs baseline: 1.1088x; 1.0430x over previous
import jax
import jax.numpy as jnp
import numpy as np
from jax import lax
from jax.experimental import pallas as pl
from jax.experimental.pallas import tpu as pltpu
from jax.experimental.pallas import tpu_sc as plsc

N = 10000
E = 160000
D_IN = 32
D_OUT = 32
D_EDGE = 16
HID = 1024
WPAD = 128
W48 = 48
CCOL = D_OUT

NC = 2
NS = 16
NW = NC * NS

ROWS_W = E // NW
CHUNK = 125
CHUNKS_W = ROWS_W // CHUNK
OUTER = 10
INNER = CHUNKS_W // OUTER


def _mesh():
    return plsc.VectorSubcoreMesh(core_axis_name="c", subcore_axis_name="s",
                                  num_cores=NC, num_subcores=NS)


def _gather_body(x_hbm, srcm_hbm, out_hbm, idx_v, row_v, sem):
    wid = lax.axis_index("s") * NC + lax.axis_index("c")
    pltpu.sync_copy(srcm_hbm.at[pl.ds(wid * CHUNKS_W, CHUNKS_W)], idx_v)

    def body(j, carry):
        pltpu.async_copy(x_hbm.at[idx_v.at[j]], row_v, sem).wait()
        pltpu.sync_copy(row_v,
                        out_hbm.at[pl.ds(wid * ROWS_W + j * CHUNK, CHUNK)])
        return carry

    lax.fori_loop(0, CHUNKS_W, body, 0)


@jax.jit
def _sc_gather(x128, src_m):
    return pl.kernel(
        _gather_body,
        out_type=jax.ShapeDtypeStruct((E, WPAD), jnp.float32),
        mesh=_mesh(),
        scratch_types=[
            pltpu.VMEM((CHUNKS_W, CHUNK), jnp.int32),
            pltpu.VMEM((CHUNK, WPAD), jnp.float32),
            pltpu.SemaphoreType.DMA,
        ],
        compiler_params=pltpu.CompilerParams(use_tc_tiling_on_sc=False),
    )(x128, src_m)


def _scatter_body(msg_hbm, dstm_hbm, zero_hbm, out0_hbm, out1_hbm,
                  idx_v, msg_v, acc_sh):
    cid = lax.axis_index("c")
    sid = lax.axis_index("s")
    wid = sid * NC + cid

    @pl.when(sid == 0)
    def _():
        pltpu.sync_copy(zero_hbm, acc_sh)

    plsc.subcore_barrier()

    pltpu.sync_copy(dstm_hbm.at[pl.ds(wid * CHUNKS_W, CHUNKS_W)], idx_v)

    def outer(c, carry):
        pltpu.sync_copy(
            msg_hbm.at[pl.ds(wid * ROWS_W + c * (INNER * CHUNK),
                             INNER * CHUNK), pl.ds(0, W48)], msg_v)

        def inner(j, carry2):
            pltpu.sync_copy(msg_v.at[pl.ds(j * CHUNK, CHUNK)],
                            acc_sh.at[idx_v.at[c * INNER + j]], add=True)
            return carry2

        lax.fori_loop(0, INNER, inner, 0)
        return carry

    lax.fori_loop(0, OUTER, outer, 0)
    plsc.subcore_barrier()

    rows0 = sid * (N // NS)

    @pl.when(cid == 0)
    def _():
        pltpu.sync_copy(acc_sh.at[pl.ds(rows0, N // NS)],
                        out0_hbm.at[pl.ds(rows0, N // NS)])

    @pl.when(cid == 1)
    def _():
        pltpu.sync_copy(acc_sh.at[pl.ds(rows0, N // NS)],
                        out1_hbm.at[pl.ds(rows0, N // NS)])


@jax.jit
def _sc_scatter(msg, dst_m, zeros_acc):
    return pl.kernel(
        _scatter_body,
        out_type=(jax.ShapeDtypeStruct((N, W48), jnp.float32),
                  jax.ShapeDtypeStruct((N, W48), jnp.float32)),
        mesh=_mesh(),
        scratch_types=[
            pltpu.VMEM((CHUNKS_W, CHUNK), jnp.int32),
            pltpu.VMEM((INNER * CHUNK, W48), jnp.float32),
            pltpu.VMEM_SHARED((N, W48), jnp.float32),
        ],
        compiler_params=pltpu.CompilerParams(use_tc_tiling_on_sc=False),
    )(msg, dst_m, zeros_acc)


TE = 2000


def _edge_body(ea_ref, xj_ref, w1_ref, b1_ref, w2_ref,
               r_ref, s_ref, b2m_ref, c_ref, out_ref):
    h = jnp.dot(ea_ref[...].astype(jnp.bfloat16), w1_ref[...],
                preferred_element_type=jnp.float32)
    h = h + b1_ref[...]
    h = jnp.maximum(h, 0.01 * h)
    w = jnp.dot(h.astype(jnp.bfloat16), w2_ref[...],
                preferred_element_type=jnp.float32)
    xjb = xj_ref[:, :D_IN].astype(jnp.bfloat16)
    xr = jnp.dot(xjb, r_ref[...],
                 preferred_element_type=jnp.float32)
    msg = jnp.dot((xr * w).astype(jnp.bfloat16), s_ref[...],
                  preferred_element_type=jnp.float32)
    out_ref[...] = msg + jnp.dot(xjb, b2m_ref[...],
                                 preferred_element_type=jnp.float32) + c_ref[...]


_R_EXPAND = np.zeros((D_IN, HID), dtype=np.float32)
for _i in range(D_IN):
    _R_EXPAND[_i, _i * D_OUT:(_i + 1) * D_OUT] = 1.0
_S_SELECT = np.zeros((HID, WPAD), dtype=np.float32)
for _i in range(D_IN):
    for _o in range(D_OUT):
        _S_SELECT[_i * D_OUT + _o, _o] = 1.0
_C_ONES = np.zeros((1, WPAD), dtype=np.float32)
_C_ONES[0, CCOL] = 1.0


@jax.jit
def _tc_edge(ea, xj, W1, b1, W2, b2):
    grid = (E // TE,)
    return pl.pallas_call(
        _edge_body,
        grid=grid,
        in_specs=[
            pl.BlockSpec((TE, D_EDGE), lambda i: (i, 0)),
            pl.BlockSpec((TE, WPAD), lambda i: (i, 0)),
            pl.BlockSpec((D_EDGE, HID), lambda i: (0, 0)),
            pl.BlockSpec((1, HID), lambda i: (0, 0)),
            pl.BlockSpec((HID, HID), lambda i: (0, 0)),
            pl.BlockSpec((D_IN, HID), lambda i: (0, 0)),
            pl.BlockSpec((HID, WPAD), lambda i: (0, 0)),
            pl.BlockSpec((D_IN, WPAD), lambda i: (0, 0)),
            pl.BlockSpec((1, WPAD), lambda i: (0, 0)),
        ],
        out_specs=pl.BlockSpec((TE, WPAD), lambda i: (i, 0)),
        out_shape=jax.ShapeDtypeStruct((E, WPAD), jnp.float32),
        compiler_params=pltpu.CompilerParams(
            dimension_semantics=("arbitrary",)),
    )(ea, xj, W1.astype(jnp.bfloat16),
      b1.reshape(1, HID),
      W2.astype(jnp.bfloat16),
      jnp.asarray(_R_EXPAND, dtype=jnp.bfloat16),
      jnp.asarray(_S_SELECT, dtype=jnp.bfloat16),
      jnp.pad(b2.reshape(D_IN, D_OUT),
              ((0, 0), (0, WPAD - D_OUT))).astype(jnp.bfloat16),
      jnp.asarray(_C_ONES))


def _finish_body(p0_ref, p1_ref, x_ref, root_ref, cb_ref, g_ref, b_ref,
                 out_ref):
    p = p0_ref[...] + p1_ref[...]
    agg = p[:, :D_OUT] / jnp.maximum(p[:, CCOL:CCOL + 1], 1.0)
    pre = agg + jnp.dot(x_ref[...], root_ref[...],
                        preferred_element_type=jnp.float32) + cb_ref[...]
    mean = jnp.mean(pre, axis=0, keepdims=True)
    cen = pre - mean
    var = jnp.mean(cen * cen, axis=0, keepdims=True)
    out_ref[...] = cen * lax.rsqrt(var + 1e-5) * g_ref[...] + b_ref[...]


@jax.jit
def _tc_finish(p0, p1, x, root, conv_bias, gamma, beta):
    return pl.pallas_call(
        _finish_body,
        out_shape=jax.ShapeDtypeStruct((N, D_OUT), jnp.float32),
    )(p0, p1, x, root, conv_bias.reshape(1, D_OUT),
      gamma.reshape(1, D_OUT), beta.reshape(1, D_OUT))


def kernel(x, edge_index, edge_attr, W1, b1, W2, b2, root, conv_bias,
           gamma, beta):
    src_m = edge_index[0].astype(jnp.int32).reshape(NW * CHUNKS_W, CHUNK)
    dst_m = edge_index[1].astype(jnp.int32).reshape(NW * CHUNKS_W, CHUNK)
    x128 = jnp.pad(x, ((0, 0), (0, WPAD - D_IN)))

    xj = _sc_gather(x128, src_m)
    msg = _tc_edge(edge_attr, xj, W1, b1, W2, b2)
    zeros_acc = jnp.zeros((N, W48), jnp.float32)
    p0, p1 = _sc_scatter(msg, dst_m, zeros_acc)
    return _tc_finish(p0, p1, x, root, conv_bias, gamma, beta)


# speedup vs baseline: 1.1471x; 1.0346x over previous
import jax
import jax.numpy as jnp
import numpy as np
from jax import lax
from jax.experimental import pallas as pl
from jax.experimental.pallas import tpu as pltpu
from jax.experimental.pallas import tpu_sc as plsc

N = 10000
E = 160000
D_IN = 32
D_OUT = 32
D_EDGE = 16
HID = 1024
WPAD = 128
W48 = 48
CCOL = D_OUT

NC = 2
NS = 16
NW = NC * NS

ROWS_W = E // NW
CHUNK = 125
CHUNKS_W = ROWS_W // CHUNK
OUTER = 10
INNER = CHUNKS_W // OUTER


def _mesh():
    return plsc.VectorSubcoreMesh(core_axis_name="c", subcore_axis_name="s",
                                  num_cores=NC, num_subcores=NS)


def _gather_body(x_hbm, srcm_hbm, out_hbm, idx_v, row_v, sem):
    wid = lax.axis_index("s") * NC + lax.axis_index("c")
    pltpu.sync_copy(srcm_hbm.at[pl.ds(wid * CHUNKS_W, CHUNKS_W)], idx_v)

    def body(j, carry):
        pltpu.async_copy(x_hbm.at[idx_v.at[j]], row_v, sem).wait()
        pltpu.sync_copy(row_v,
                        out_hbm.at[pl.ds(wid * ROWS_W + j * CHUNK, CHUNK)])
        return carry

    lax.fori_loop(0, CHUNKS_W, body, 0)


@jax.jit
def _sc_gather(x, src_m):
    return pl.kernel(
        _gather_body,
        out_type=jax.ShapeDtypeStruct((E, D_IN), jnp.float32),
        mesh=_mesh(),
        scratch_types=[
            pltpu.VMEM((CHUNKS_W, CHUNK), jnp.int32),
            pltpu.VMEM((CHUNK, D_IN), jnp.float32),
            pltpu.SemaphoreType.DMA,
        ],
        compiler_params=pltpu.CompilerParams(use_tc_tiling_on_sc=False),
    )(x, src_m)


def _scatter_body(msg_hbm, dstm_hbm, zero_hbm, out0_hbm, out1_hbm,
                  idx_v, msg_v, acc_sh):
    cid = lax.axis_index("c")
    sid = lax.axis_index("s")
    wid = sid * NC + cid

    @pl.when(sid == 0)
    def _():
        pltpu.sync_copy(zero_hbm, acc_sh)

    plsc.subcore_barrier()

    pltpu.sync_copy(dstm_hbm.at[pl.ds(wid * CHUNKS_W, CHUNKS_W)], idx_v)

    def outer(c, carry):
        pltpu.sync_copy(
            msg_hbm.at[pl.ds(wid * ROWS_W + c * (INNER * CHUNK),
                             INNER * CHUNK), pl.ds(0, W48)], msg_v)

        def inner(j, carry2):
            pltpu.sync_copy(msg_v.at[pl.ds(j * CHUNK, CHUNK)],
                            acc_sh.at[idx_v.at[c * INNER + j]], add=True)
            return carry2

        lax.fori_loop(0, INNER, inner, 0)
        return carry

    lax.fori_loop(0, OUTER, outer, 0)
    plsc.subcore_barrier()

    rows0 = sid * (N // NS)

    @pl.when(cid == 0)
    def _():
        pltpu.sync_copy(acc_sh.at[pl.ds(rows0, N // NS)],
                        out0_hbm.at[pl.ds(rows0, N // NS)])

    @pl.when(cid == 1)
    def _():
        pltpu.sync_copy(acc_sh.at[pl.ds(rows0, N // NS)],
                        out1_hbm.at[pl.ds(rows0, N // NS)])


@jax.jit
def _sc_scatter(msg, dst_m, zeros_acc):
    return pl.kernel(
        _scatter_body,
        out_type=(jax.ShapeDtypeStruct((N, W48), jnp.float32),
                  jax.ShapeDtypeStruct((N, W48), jnp.float32)),
        mesh=_mesh(),
        scratch_types=[
            pltpu.VMEM((CHUNKS_W, CHUNK), jnp.int32),
            pltpu.VMEM((INNER * CHUNK, W48), jnp.float32),
            pltpu.VMEM_SHARED((N, W48), jnp.float32),
        ],
        compiler_params=pltpu.CompilerParams(use_tc_tiling_on_sc=False),
    )(msg, dst_m, zeros_acc)


TE = 1600


def _edge_body(ea_ref, xj_ref, w1_ref, b1_ref, w2_ref,
               r_ref, s_ref, b2m_ref, c_ref, out_ref):
    h = jnp.dot(ea_ref[...].astype(jnp.bfloat16), w1_ref[...],
                preferred_element_type=jnp.float32)
    h = h + b1_ref[...]
    h = jnp.maximum(h, 0.01 * h)
    w = jnp.dot(h.astype(jnp.bfloat16), w2_ref[...],
                preferred_element_type=jnp.float32)
    p = xj_ref[...]
    xjb = jnp.concatenate(
        [p[:, q * D_IN:(q + 1) * D_IN] for q in range(4)],
        axis=0).astype(jnp.bfloat16)
    xr = jnp.dot(xjb, r_ref[...],
                 preferred_element_type=jnp.float32)
    msg = jnp.dot((xr * w).astype(jnp.bfloat16), s_ref[...],
                  preferred_element_type=jnp.float32)
    out_ref[...] = msg + jnp.dot(xjb, b2m_ref[...],
                                 preferred_element_type=jnp.float32) + c_ref[...]


_R_EXPAND = np.zeros((D_IN, HID), dtype=np.float32)
for _i in range(D_IN):
    _R_EXPAND[_i, _i * D_OUT:(_i + 1) * D_OUT] = 1.0
_S_SELECT = np.zeros((HID, WPAD), dtype=np.float32)
for _i in range(D_IN):
    for _o in range(D_OUT):
        _S_SELECT[_i * D_OUT + _o, _o] = 1.0
_C_ONES = np.zeros((1, WPAD), dtype=np.float32)
_C_ONES[0, CCOL] = 1.0


@jax.jit
def _tc_edge(ea, xj, W1, b1, W2, b2):
    grid = (E // TE,)
    return pl.pallas_call(
        _edge_body,
        grid=grid,
        in_specs=[
            pl.BlockSpec((TE, D_EDGE), lambda i: (i, 0)),
            pl.BlockSpec((TE // 4, WPAD), lambda i: (i, 0)),
            pl.BlockSpec((D_EDGE, HID), lambda i: (0, 0)),
            pl.BlockSpec((1, HID), lambda i: (0, 0)),
            pl.BlockSpec((HID, HID), lambda i: (0, 0)),
            pl.BlockSpec((D_IN, HID), lambda i: (0, 0)),
            pl.BlockSpec((HID, WPAD), lambda i: (0, 0)),
            pl.BlockSpec((D_IN, WPAD), lambda i: (0, 0)),
            pl.BlockSpec((1, WPAD), lambda i: (0, 0)),
        ],
        out_specs=pl.BlockSpec((TE, WPAD), lambda i: (i, 0)),
        out_shape=jax.ShapeDtypeStruct((E, WPAD), jnp.float32),
        compiler_params=pltpu.CompilerParams(
            dimension_semantics=("arbitrary",)),
    )(ea, xj, W1.astype(jnp.bfloat16),
      b1.reshape(1, HID),
      W2.astype(jnp.bfloat16),
      jnp.asarray(_R_EXPAND, dtype=jnp.bfloat16),
      jnp.asarray(_S_SELECT, dtype=jnp.bfloat16),
      jnp.pad(b2.reshape(D_IN, D_OUT),
              ((0, 0), (0, WPAD - D_OUT))).astype(jnp.bfloat16),
      jnp.asarray(_C_ONES))


def _finish_body(p0_ref, p1_ref, x_ref, root_ref, cb_ref, g_ref, b_ref,
                 out_ref):
    p = p0_ref[...] + p1_ref[...]
    agg = p[:, :D_OUT] / jnp.maximum(p[:, CCOL:CCOL + 1], 1.0)
    pre = agg + jnp.dot(x_ref[...], root_ref[...],
                        preferred_element_type=jnp.float32) + cb_ref[...]
    mean = jnp.mean(pre, axis=0, keepdims=True)
    cen = pre - mean
    var = jnp.mean(cen * cen, axis=0, keepdims=True)
    out_ref[...] = cen * lax.rsqrt(var + 1e-5) * g_ref[...] + b_ref[...]


@jax.jit
def _tc_finish(p0, p1, x, root, conv_bias, gamma, beta):
    return pl.pallas_call(
        _finish_body,
        out_shape=jax.ShapeDtypeStruct((N, D_OUT), jnp.float32),
    )(p0, p1, x, root, conv_bias.reshape(1, D_OUT),
      gamma.reshape(1, D_OUT), beta.reshape(1, D_OUT))


def kernel(x, edge_index, edge_attr, W1, b1, W2, b2, root, conv_bias,
           gamma, beta):
    src = edge_index[0].astype(jnp.int32)
    dst_m = edge_index[1].astype(jnp.int32).reshape(NW * CHUNKS_W, CHUNK)

    src_g = (src.reshape(E // TE, 4, TE // 4)
             .transpose(0, 2, 1).reshape(NW * CHUNKS_W, CHUNK))

    xj = _sc_gather(x, src_g)
    msg = _tc_edge(edge_attr, xj.reshape(E // 4, WPAD), W1, b1, W2, b2)
    zeros_acc = jnp.zeros((N, W48), jnp.float32)
    p0, p1 = _sc_scatter(msg, dst_m, zeros_acc)
    return _tc_finish(p0, p1, x, root, conv_bias, gamma, beta)


# speedup vs baseline: 1.1726x; 1.0222x over previous
import jax
import jax.numpy as jnp
import numpy as np
from jax import lax
from jax.experimental import pallas as pl
from jax.experimental.pallas import tpu as pltpu
from jax.experimental.pallas import tpu_sc as plsc

N = 10000
E = 160000
D_IN = 32
D_OUT = 32
D_EDGE = 16
HID = 1024
WPAD = 128
W48 = 48
CCOL = D_OUT

NC = 2
NS = 16
NW = NC * NS

ROWS_W = E // NW
CHUNK = 125
CHUNKS_W = ROWS_W // CHUNK
OUTER = 10
INNER = CHUNKS_W // OUTER


def _mesh():
    return plsc.VectorSubcoreMesh(core_axis_name="c", subcore_axis_name="s",
                                  num_cores=NC, num_subcores=NS)


def _gather_body(x_hbm, srcm_hbm, out_hbm, idx_v, row_v, sem):
    wid = lax.axis_index("s") * NC + lax.axis_index("c")
    pltpu.sync_copy(srcm_hbm.at[pl.ds(wid * CHUNKS_W, CHUNKS_W)], idx_v)

    def body(j, carry):
        pltpu.async_copy(x_hbm.at[idx_v.at[j]], row_v, sem).wait()
        pltpu.sync_copy(row_v,
                        out_hbm.at[pl.ds(wid * ROWS_W + j * CHUNK, CHUNK)])
        return carry

    lax.fori_loop(0, CHUNKS_W, body, 0)


@jax.jit
def _sc_gather(x, src_m):
    return pl.kernel(
        _gather_body,
        out_type=jax.ShapeDtypeStruct((E, D_IN), jnp.float32),
        mesh=_mesh(),
        scratch_types=[
            pltpu.VMEM((CHUNKS_W, CHUNK), jnp.int32),
            pltpu.VMEM((CHUNK, D_IN), jnp.float32),
            pltpu.SemaphoreType.DMA,
        ],
        compiler_params=pltpu.CompilerParams(use_tc_tiling_on_sc=False),
    )(x, src_m)


def _scatter_body(msg_hbm, dstm_hbm, zero_hbm, out0_hbm, out1_hbm,
                  idx_v, msg_v, acc_sh):
    cid = lax.axis_index("c")
    sid = lax.axis_index("s")
    wid = sid * NC + cid

    @pl.when(sid == 0)
    def _():
        pltpu.sync_copy(zero_hbm, acc_sh)

    plsc.subcore_barrier()

    pltpu.sync_copy(dstm_hbm.at[pl.ds(wid * CHUNKS_W, CHUNKS_W)], idx_v)

    def outer(c, carry):
        pltpu.sync_copy(
            msg_hbm.at[pl.ds(wid * ROWS_W + c * (INNER * CHUNK),
                             INNER * CHUNK), pl.ds(0, W48)], msg_v)

        def inner(j, carry2):
            pltpu.sync_copy(msg_v.at[pl.ds(j * CHUNK, CHUNK)],
                            acc_sh.at[idx_v.at[c * INNER + j]], add=True)
            return carry2

        lax.fori_loop(0, INNER, inner, 0)
        return carry

    lax.fori_loop(0, OUTER, outer, 0)
    plsc.subcore_barrier()

    rows0 = sid * (N // NS)

    @pl.when(cid == 0)
    def _():
        pltpu.sync_copy(acc_sh.at[pl.ds(rows0, N // NS)],
                        out0_hbm.at[pl.ds(rows0, N // NS), pl.ds(0, W48)])

    @pl.when(cid == 1)
    def _():
        pltpu.sync_copy(acc_sh.at[pl.ds(rows0, N // NS)],
                        out1_hbm.at[pl.ds(rows0, N // NS), pl.ds(0, W48)])


@jax.jit
def _sc_scatter(msg, dst_m, zeros_acc):
    return pl.kernel(
        _scatter_body,
        out_type=(jax.ShapeDtypeStruct((N, WPAD), jnp.float32),
                  jax.ShapeDtypeStruct((N, WPAD), jnp.float32)),
        mesh=_mesh(),
        scratch_types=[
            pltpu.VMEM((CHUNKS_W, CHUNK), jnp.int32),
            pltpu.VMEM((INNER * CHUNK, W48), jnp.float32),
            pltpu.VMEM_SHARED((N, W48), jnp.float32),
        ],
        compiler_params=pltpu.CompilerParams(use_tc_tiling_on_sc=False),
    )(msg, dst_m, zeros_acc)


TE = 1600


def _edge_body(ea_ref, xj_ref, w1_ref, b1_ref, w2_ref,
               r_ref, s_ref, b2m_ref, c_ref, out_ref):
    h = jnp.dot(ea_ref[...].astype(jnp.bfloat16), w1_ref[...],
                preferred_element_type=jnp.float32)
    h = h + b1_ref[...]
    h = jnp.maximum(h, 0.01 * h)
    w = jnp.dot(h.astype(jnp.bfloat16), w2_ref[...],
                preferred_element_type=jnp.float32)
    p = xj_ref[...]
    xjb = jnp.concatenate(
        [p[:, q * D_IN:(q + 1) * D_IN] for q in range(4)],
        axis=0).astype(jnp.bfloat16)
    xr = jnp.dot(xjb, r_ref[...],
                 preferred_element_type=jnp.float32)
    msg = jnp.dot((xr * w).astype(jnp.bfloat16), s_ref[...],
                  preferred_element_type=jnp.float32)
    out_ref[...] = msg + jnp.dot(xjb, b2m_ref[...],
                                 preferred_element_type=jnp.float32) + c_ref[...]


_R_EXPAND = np.zeros((D_IN, HID), dtype=np.float32)
for _i in range(D_IN):
    _R_EXPAND[_i, _i * D_OUT:(_i + 1) * D_OUT] = 1.0
_S_SELECT = np.zeros((HID, WPAD), dtype=np.float32)
for _i in range(D_IN):
    for _o in range(D_OUT):
        _S_SELECT[_i * D_OUT + _o, _o] = 1.0
_C_ONES = np.zeros((1, WPAD), dtype=np.float32)
_C_ONES[0, CCOL] = 1.0

_SRC_PERM = (np.arange(E, dtype=np.int32)
             .reshape(E // TE, 4, TE // 4)
             .transpose(0, 2, 1).reshape(E))


@jax.jit
def _tc_edge(ea, xj, W1, b1, W2, b2):
    grid = (E // TE,)
    return pl.pallas_call(
        _edge_body,
        grid=grid,
        in_specs=[
            pl.BlockSpec((TE, D_EDGE), lambda i: (i, 0)),
            pl.BlockSpec((TE // 4, WPAD), lambda i: (i, 0)),
            pl.BlockSpec((D_EDGE, HID), lambda i: (0, 0)),
            pl.BlockSpec((1, HID), lambda i: (0, 0)),
            pl.BlockSpec((HID, HID), lambda i: (0, 0)),
            pl.BlockSpec((D_IN, HID), lambda i: (0, 0)),
            pl.BlockSpec((HID, WPAD), lambda i: (0, 0)),
            pl.BlockSpec((D_IN, WPAD), lambda i: (0, 0)),
            pl.BlockSpec((1, WPAD), lambda i: (0, 0)),
        ],
        out_specs=pl.BlockSpec((TE, WPAD), lambda i: (i, 0)),
        out_shape=jax.ShapeDtypeStruct((E, WPAD), jnp.float32),
        compiler_params=pltpu.CompilerParams(
            dimension_semantics=("arbitrary",)),
    )(ea, xj, W1.astype(jnp.bfloat16),
      b1.reshape(1, HID),
      W2.astype(jnp.bfloat16),
      jnp.asarray(_R_EXPAND, dtype=jnp.bfloat16),
      jnp.asarray(_S_SELECT, dtype=jnp.bfloat16),
      jnp.pad(b2.reshape(D_IN, D_OUT),
              ((0, 0), (0, WPAD - D_OUT))).astype(jnp.bfloat16),
      jnp.asarray(_C_ONES))


def _finish_body(p0_ref, p1_ref, x_ref, root_ref, cb_ref, g_ref, b_ref,
                 out_ref):
    p = p0_ref[...] + p1_ref[...]
    agg = p[:, :D_OUT] / jnp.maximum(p[:, CCOL:CCOL + 1], 1.0)
    pre = agg + jnp.dot(x_ref[...], root_ref[...],
                        preferred_element_type=jnp.float32) + cb_ref[...]
    mean = jnp.mean(pre, axis=0, keepdims=True)
    cen = pre - mean
    var = jnp.mean(cen * cen, axis=0, keepdims=True)
    out_ref[...] = cen * lax.rsqrt(var + 1e-5) * g_ref[...] + b_ref[...]


@jax.jit
def _tc_finish(p0, p1, x, root, conv_bias, gamma, beta):
    return pl.pallas_call(
        _finish_body,
        out_shape=jax.ShapeDtypeStruct((N, D_OUT), jnp.float32),
    )(p0, p1, x, root, conv_bias.reshape(1, D_OUT),
      gamma.reshape(1, D_OUT), beta.reshape(1, D_OUT))


def kernel(x, edge_index, edge_attr, W1, b1, W2, b2, root, conv_bias,
           gamma, beta):
    src = edge_index[0].astype(jnp.int32)
    dst_m = edge_index[1].astype(jnp.int32).reshape(NW * CHUNKS_W, CHUNK)

    src_g = jnp.take(src, jnp.asarray(_SRC_PERM),
                     axis=0).reshape(NW * CHUNKS_W, CHUNK)

    xj = _sc_gather(x, src_g)
    msg = _tc_edge(edge_attr, xj.reshape(E // 4, WPAD), W1, b1, W2, b2)
    zeros_acc = jnp.zeros((N, W48), jnp.float32)
    p0, p1 = _sc_scatter(msg, dst_m, zeros_acc)
    return _tc_finish(p0, p1, x, root, conv_bias, gamma, beta)


# speedup vs baseline: 1.1891x; 1.0141x over previous
import jax
import jax.numpy as jnp
import numpy as np
from jax import lax
from jax.experimental import pallas as pl
from jax.experimental.pallas import tpu as pltpu
from jax.experimental.pallas import tpu_sc as plsc

N = 10000
E = 160000
D_IN = 32
D_OUT = 32
D_EDGE = 16
HID = 1024
WPAD = 128
W48 = 48
CCOL = D_OUT

NC = 2
NS = 16
NW = NC * NS

ROWS_W = E // NW
CHUNK = 125
CHUNKS_W = ROWS_W // CHUNK
OUTER = 10
INNER = CHUNKS_W // OUTER


def _mesh():
    return plsc.VectorSubcoreMesh(core_axis_name="c", subcore_axis_name="s",
                                  num_cores=NC, num_subcores=NS)


def _gather_body(x_hbm, srcm_hbm, out_hbm, idx_v, row_v, sem):
    wid = lax.axis_index("s") * NC + lax.axis_index("c")
    pltpu.sync_copy(srcm_hbm.at[pl.ds(wid * CHUNKS_W, CHUNKS_W)], idx_v)

    def body(j, carry):
        pltpu.async_copy(x_hbm.at[idx_v.at[j]], row_v, sem).wait()
        pltpu.sync_copy(row_v,
                        out_hbm.at[pl.ds(wid * ROWS_W + j * CHUNK, CHUNK)])
        return carry

    lax.fori_loop(0, CHUNKS_W, body, 0)


@jax.jit
def _sc_gather(x, src_m):
    return pl.kernel(
        _gather_body,
        out_type=jax.ShapeDtypeStruct((E, D_IN), jnp.float32),
        mesh=_mesh(),
        scratch_types=[
            pltpu.VMEM((CHUNKS_W, CHUNK), jnp.int32),
            pltpu.VMEM((CHUNK, D_IN), jnp.float32),
            pltpu.SemaphoreType.DMA,
        ],
        compiler_params=pltpu.CompilerParams(use_tc_tiling_on_sc=False),
    )(x, src_m)


def _scatter_body(msg_hbm, dstm_hbm, zero_hbm, out0_hbm, out1_hbm,
                  idx_v, msg_v, acc_sh):
    cid = lax.axis_index("c")
    sid = lax.axis_index("s")
    wid = sid * NC + cid

    @pl.when(sid == 0)
    def _():
        pltpu.sync_copy(zero_hbm, acc_sh)

    plsc.subcore_barrier()

    pltpu.sync_copy(dstm_hbm.at[pl.ds(wid * CHUNKS_W, CHUNKS_W)], idx_v)

    def outer(c, carry):
        pltpu.sync_copy(
            msg_hbm.at[pl.ds(wid * ROWS_W + c * (INNER * CHUNK),
                             INNER * CHUNK), pl.ds(0, W48)], msg_v)

        def inner(j, carry2):
            pltpu.sync_copy(msg_v.at[pl.ds(j * CHUNK, CHUNK)],
                            acc_sh.at[idx_v.at[c * INNER + j]], add=True)
            return carry2

        lax.fori_loop(0, INNER, inner, 0)
        return carry

    lax.fori_loop(0, OUTER, outer, 0)
    plsc.subcore_barrier()

    rows0 = sid * (N // NS)

    @pl.when(cid == 0)
    def _():
        pltpu.sync_copy(acc_sh.at[pl.ds(rows0, N // NS)],
                        out0_hbm.at[pl.ds(rows0, N // NS), pl.ds(0, W48)])

    @pl.when(cid == 1)
    def _():
        pltpu.sync_copy(acc_sh.at[pl.ds(rows0, N // NS)],
                        out1_hbm.at[pl.ds(rows0, N // NS), pl.ds(0, W48)])


@jax.jit
def _sc_scatter(msg, dst_m, zeros_acc):
    return pl.kernel(
        _scatter_body,
        out_type=(jax.ShapeDtypeStruct((N, WPAD), jnp.float32),
                  jax.ShapeDtypeStruct((N, WPAD), jnp.float32)),
        mesh=_mesh(),
        scratch_types=[
            pltpu.VMEM((CHUNKS_W, CHUNK), jnp.int32),
            pltpu.VMEM((INNER * CHUNK, W48), jnp.float32),
            pltpu.VMEM_SHARED((N, W48), jnp.float32),
        ],
        compiler_params=pltpu.CompilerParams(use_tc_tiling_on_sc=False),
    )(msg, dst_m, zeros_acc)


TE = 3200


def _edge_body(ea_ref, xj_ref, w1_ref, b1_ref, w2_ref,
               r_ref, s_ref, b2m_ref, c_ref, out_ref):
    h = jnp.dot(ea_ref[...].astype(jnp.bfloat16), w1_ref[...],
                preferred_element_type=jnp.float32)
    h = h + b1_ref[...]
    h = jnp.maximum(h, 0.01 * h)
    w = jnp.dot(h.astype(jnp.bfloat16), w2_ref[...],
                preferred_element_type=jnp.float32)
    p = xj_ref[...]
    xjb = jnp.concatenate(
        [p[:, q * D_IN:(q + 1) * D_IN] for q in range(4)],
        axis=0).astype(jnp.bfloat16)
    xr = jnp.dot(xjb, r_ref[...],
                 preferred_element_type=jnp.float32)
    msg = jnp.dot((xr * w).astype(jnp.bfloat16), s_ref[...],
                  preferred_element_type=jnp.float32)
    out_ref[...] = msg + jnp.dot(xjb, b2m_ref[...],
                                 preferred_element_type=jnp.float32) + c_ref[...]


_R_EXPAND = np.zeros((D_IN, HID), dtype=np.float32)
for _i in range(D_IN):
    _R_EXPAND[_i, _i * D_OUT:(_i + 1) * D_OUT] = 1.0
_S_SELECT = np.zeros((HID, WPAD), dtype=np.float32)
for _i in range(D_IN):
    for _o in range(D_OUT):
        _S_SELECT[_i * D_OUT + _o, _o] = 1.0
_C_ONES = np.zeros((1, WPAD), dtype=np.float32)
_C_ONES[0, CCOL] = 1.0

_SRC_PERM = (np.arange(E, dtype=np.int32)
             .reshape(E // TE, 4, TE // 4)
             .transpose(0, 2, 1).reshape(E))


@jax.jit
def _tc_edge(ea, xj, W1, b1, W2, b2):
    grid = (E // TE,)
    return pl.pallas_call(
        _edge_body,
        grid=grid,
        in_specs=[
            pl.BlockSpec((TE, D_EDGE), lambda i: (i, 0)),
            pl.BlockSpec((TE // 4, WPAD), lambda i: (i, 0)),
            pl.BlockSpec((D_EDGE, HID), lambda i: (0, 0)),
            pl.BlockSpec((1, HID), lambda i: (0, 0)),
            pl.BlockSpec((HID, HID), lambda i: (0, 0)),
            pl.BlockSpec((D_IN, HID), lambda i: (0, 0)),
            pl.BlockSpec((HID, WPAD), lambda i: (0, 0)),
            pl.BlockSpec((D_IN, WPAD), lambda i: (0, 0)),
            pl.BlockSpec((1, WPAD), lambda i: (0, 0)),
        ],
        out_specs=pl.BlockSpec((TE, WPAD), lambda i: (i, 0)),
        out_shape=jax.ShapeDtypeStruct((E, WPAD), jnp.float32),
        compiler_params=pltpu.CompilerParams(
            dimension_semantics=("arbitrary",)),
    )(ea, xj, W1.astype(jnp.bfloat16),
      b1.reshape(1, HID),
      W2.astype(jnp.bfloat16),
      jnp.asarray(_R_EXPAND, dtype=jnp.bfloat16),
      jnp.asarray(_S_SELECT, dtype=jnp.bfloat16),
      jnp.pad(b2.reshape(D_IN, D_OUT),
              ((0, 0), (0, WPAD - D_OUT))).astype(jnp.bfloat16),
      jnp.asarray(_C_ONES))


def _finish_body(p0_ref, p1_ref, x_ref, root_ref, cb_ref, g_ref, b_ref,
                 out_ref):
    p = p0_ref[...] + p1_ref[...]
    agg = p[:, :D_OUT] / jnp.maximum(p[:, CCOL:CCOL + 1], 1.0)
    pre = agg + jnp.dot(x_ref[...], root_ref[...],
                        preferred_element_type=jnp.float32) + cb_ref[...]
    mean = jnp.mean(pre, axis=0, keepdims=True)
    cen = pre - mean
    var = jnp.mean(cen * cen, axis=0, keepdims=True)
    out_ref[...] = cen * lax.rsqrt(var + 1e-5) * g_ref[...] + b_ref[...]


@jax.jit
def _tc_finish(p0, p1, x, root, conv_bias, gamma, beta):
    return pl.pallas_call(
        _finish_body,
        out_shape=jax.ShapeDtypeStruct((N, D_OUT), jnp.float32),
    )(p0, p1, x, root, conv_bias.reshape(1, D_OUT),
      gamma.reshape(1, D_OUT), beta.reshape(1, D_OUT))


def kernel(x, edge_index, edge_attr, W1, b1, W2, b2, root, conv_bias,
           gamma, beta):
    src = edge_index[0].astype(jnp.int32)
    dst_m = edge_index[1].astype(jnp.int32).reshape(NW * CHUNKS_W, CHUNK)

    src_g = jnp.take(src, jnp.asarray(_SRC_PERM),
                     axis=0).reshape(NW * CHUNKS_W, CHUNK)

    xj = _sc_gather(x, src_g)
    msg = _tc_edge(edge_attr, xj.reshape(E // 4, WPAD), W1, b1, W2, b2)
    zeros_acc = jnp.zeros((N, W48), jnp.float32)
    p0, p1 = _sc_scatter(msg, dst_m, zeros_acc)
    return _tc_finish(p0, p1, x, root, conv_bias, gamma, beta)
